# baseline (speedup 1.0000x reference)
import functools

import jax
import jax.numpy as jnp
from jax.experimental import pallas as pl
from jax.experimental.pallas import tpu as pltpu

_SIGMA = 0.1
_INV2SIG2 = 1.0 / (2.0 * _SIGMA * _SIGMA)


def _dot(a, b):
    return jax.lax.dot_general(
        a, b, (((1,), (0,)), ((), ())),
        precision=jax.lax.Precision.DEFAULT,
        preferred_element_type=jnp.float32)


def _fused_body(x_ref, w1_ref, b1_ref, w2_ref, b2_ref, ow1_ref, ow2_ref,
                ob_ref, p_ref, o_ref):
    xb = x_ref[...]
    h = jnp.maximum(_dot(xb, w1_ref[...]) + b1_ref[...], 0.0)
    v = _dot(h, w2_ref[...]) + b2_ref[...]

    tv = p_ref[0:1, :]
    gb = p_ref[1:2, :]
    gc = p_ref[2:3, :]
    lw = p_ref[3:4, :]
    lb = p_ref[4:5, :]
    c0 = p_ref[5:6, :]
    c1 = p_ref[6:7, :]
    ra = p_ref[7:8, :]

    tri = jnp.maximum(v - jnp.abs(v - tv), 0.0)
    gau = jnp.exp((gb - (2.0 * _INV2SIG2) * v) * v + gc)
    lin = v * lw + lb
    dd = jnp.abs(v - c0) + jnp.abs(v - c1)
    da = jnp.abs(ra - dd)
    rat = (da - dd) / ((1.0 + dd) * (1.0 + da))
    acts = (tri + gau) + (lin + rat)

    o = _dot(xb, ow1_ref[...]) + _dot(acts, ow2_ref[...]) + ob_ref[...]
    o_ref[...] = jnp.maximum(o, 0.0)


@functools.partial(jax.jit, static_argnames=())
def kernel(x, edge_index, edge_slices, W1, b1, W2, b2, t_param, mu, line_W,
           line_b, c_param, r_param, out_W, out_b):
    del edge_index, edge_slices
    f32 = jnp.float32
    N, D = x.shape
    H = W1.shape[1]
    F = W2.shape[1]
    CF = t_param.shape[0]
    OUT = out_W.shape[1]
    K = 4 * CF
    A = F * K
    AP = 128

    W2R = jnp.pad(jnp.repeat(W2, K, axis=1), ((0, 0), (0, AP - A)))
    b2R = jnp.pad(jnp.repeat(b2, K), (0, AP - A)).reshape(1, AP)

    zc = jnp.zeros((CF,), f32)

    def col(a, b, c, d, fill=0.0):
        base = jnp.full((K,), fill, f32)
        base = base.at[0:CF].set(a).at[CF:2 * CF].set(b)
        base = base.at[2 * CF:3 * CF].set(c).at[3 * CF:4 * CF].set(d)
        return jnp.pad(jnp.tile(base, F), (0, AP - A), constant_values=fill)

    gb = 2.0 * _INV2SIG2 * (mu[:, 0] + mu[:, 1])
    gc = -_INV2SIG2 * (mu[:, 0] ** 2 + mu[:, 1] ** 2)
    neg_huge = jnp.full((CF,), -1e30, f32)
    P = jnp.stack([
        col(t_param, neg_huge, neg_huge, neg_huge, fill=-1e30),
        col(zc, gb, zc, zc),
        col(jnp.full((CF,), -1e38, f32), gc,
            jnp.full((CF,), -1e38, f32),
            jnp.full((CF,), -1e38, f32), fill=-1e38),
        col(zc, zc, line_W[0] + line_W[1], zc),
        col(zc, zc, line_b, zc),
        col(zc, zc, zc, c_param[:, 0]),
        col(zc, zc, zc, c_param[:, 1]),
        col(zc, zc, zc, jnp.full((CF,), jnp.abs(r_param[0]), f32)),
    ])

    oW1 = out_W[:D]
    oW2 = jnp.pad(out_W[D:], ((0, AP - A), (0, 0)))
    b1r = b1.reshape(1, H)
    obr = out_b.reshape(1, OUT)

    B = 8000
    grid = pl.cdiv(N, B)
    rep = lambda i: (0, 0)

    return pl.pallas_call(
        _fused_body,
        grid=(grid,),
        in_specs=[
            pl.BlockSpec((B, D), lambda i: (i, 0)),
            pl.BlockSpec((D, H), rep),
            pl.BlockSpec((1, H), rep),
            pl.BlockSpec((H, AP), rep),
            pl.BlockSpec((1, AP), rep),
            pl.BlockSpec((D, OUT), rep),
            pl.BlockSpec((AP, OUT), rep),
            pl.BlockSpec((1, OUT), rep),
            pl.BlockSpec((8, AP), rep),
        ],
        out_specs=pl.BlockSpec((B, OUT), lambda i: (i, 0)),
        out_shape=jax.ShapeDtypeStruct((N, OUT), f32),
        compiler_params=pltpu.CompilerParams(
            dimension_semantics=("arbitrary",)),
    )(x, W1, b1r, W2R, b2R, oW1, oW2, obr, P)


# speedup vs baseline: 1.0336x; 1.0336x over previous
import functools

import jax
import jax.numpy as jnp
from jax.experimental import pallas as pl
from jax.experimental.pallas import tpu as pltpu

_SIGMA = 0.1
_INV2SIG2 = 1.0 / (2.0 * _SIGMA * _SIGMA)


def _dot(a, b):
    return jax.lax.dot_general(
        a, b, (((1,), (0,)), ((), ())),
        precision=jax.lax.Precision.DEFAULT,
        preferred_element_type=jnp.float32)


def _fused_body(x_ref, w1_ref, b1_ref, w2_ref, b2_ref, ow1_ref, ow2_ref,
                ob_ref, p_ref, o_ref):
    xb = x_ref[...]
    h = jnp.maximum(_dot(xb, w1_ref[...]) + b1_ref[...], 0.0)
    v = _dot(h, w2_ref[...]) + b2_ref[...]

    tv = p_ref[0:1, :]
    gb = p_ref[1:2, :]
    gc = p_ref[2:3, :]
    lw = p_ref[3:4, :]
    lb = p_ref[4:5, :]
    c0 = p_ref[5:6, :]
    c1 = p_ref[6:7, :]
    ra = p_ref[7:8, :]

    tri = jnp.maximum(v - jnp.abs(v - tv), 0.0)
    gau = jnp.exp((gb - (2.0 * _INV2SIG2) * v) * v + gc)
    lin = v * lw + lb
    dd = jnp.abs(v - c0) + jnp.abs(v - c1)
    da = jnp.abs(ra - dd)
    rat = (da - dd) / ((1.0 + dd) * (1.0 + da))
    acts = (tri + gau) + (lin + rat)

    o = _dot(xb, ow1_ref[...]) + _dot(acts, ow2_ref[...]) + ob_ref[...]
    o_ref[...] = jnp.maximum(o, 0.0)


@functools.partial(jax.jit, static_argnames=())
def kernel(x, edge_index, edge_slices, W1, b1, W2, b2, t_param, mu, line_W,
           line_b, c_param, r_param, out_W, out_b):
    del edge_index, edge_slices
    f32 = jnp.float32
    N, D = x.shape
    H = W1.shape[1]
    F = W2.shape[1]
    CF = t_param.shape[0]
    OUT = out_W.shape[1]
    K = 4 * CF
    A = F * K
    AP = 128

    W2R = jnp.pad(jnp.repeat(W2, K, axis=1), ((0, 0), (0, AP - A)))
    b2R = jnp.pad(jnp.repeat(b2, K), (0, AP - A)).reshape(1, AP)

    zc = jnp.zeros((CF,), f32)

    def col(a, b, c, d, fill=0.0):
        base = jnp.full((K,), fill, f32)
        base = base.at[0:CF].set(a).at[CF:2 * CF].set(b)
        base = base.at[2 * CF:3 * CF].set(c).at[3 * CF:4 * CF].set(d)
        return jnp.pad(jnp.tile(base, F), (0, AP - A), constant_values=fill)

    gb = 2.0 * _INV2SIG2 * (mu[:, 0] + mu[:, 1])
    gc = -_INV2SIG2 * (mu[:, 0] ** 2 + mu[:, 1] ** 2)
    neg_huge = jnp.full((CF,), -1e30, f32)
    P = jnp.stack([
        col(t_param, neg_huge, neg_huge, neg_huge, fill=-1e30),
        col(zc, gb, zc, zc),
        col(jnp.full((CF,), -1e38, f32), gc,
            jnp.full((CF,), -1e38, f32),
            jnp.full((CF,), -1e38, f32), fill=-1e38),
        col(zc, zc, line_W[0] + line_W[1], zc),
        col(zc, zc, line_b, zc),
        col(zc, zc, zc, c_param[:, 0]),
        col(zc, zc, zc, c_param[:, 1]),
        col(zc, zc, zc, jnp.full((CF,), jnp.abs(r_param[0]), f32)),
    ])

    oW1 = out_W[:D]
    oW2 = jnp.pad(out_W[D:], ((0, AP - A), (0, 0)))
    b1r = b1.reshape(1, H)
    obr = out_b.reshape(1, OUT)

    B = 4000
    grid = pl.cdiv(N, B)
    rep = lambda i: (0, 0)

    return pl.pallas_call(
        _fused_body,
        grid=(grid,),
        in_specs=[
            pl.BlockSpec((B, D), lambda i: (i, 0)),
            pl.BlockSpec((D, H), rep),
            pl.BlockSpec((1, H), rep),
            pl.BlockSpec((H, AP), rep),
            pl.BlockSpec((1, AP), rep),
            pl.BlockSpec((D, OUT), rep),
            pl.BlockSpec((AP, OUT), rep),
            pl.BlockSpec((1, OUT), rep),
            pl.BlockSpec((8, AP), rep),
        ],
        out_specs=pl.BlockSpec((B, OUT), lambda i: (i, 0)),
        out_shape=jax.ShapeDtypeStruct((N, OUT), f32),
        compiler_params=pltpu.CompilerParams(
            dimension_semantics=("arbitrary",)),
    )(x, W1, b1r, W2R, b2R, oW1, oW2, obr, P)


# speedup vs baseline: 1.2635x; 1.2224x over previous
import functools

import jax
import jax.numpy as jnp
from jax.experimental import pallas as pl
from jax.experimental.pallas import tpu as pltpu

_SIGMA = 0.1
_INV2SIG2 = 1.0 / (2.0 * _SIGMA * _SIGMA)


def _dot(a, b):
    return jax.lax.dot_general(
        a, b, (((1,), (0,)), ((), ())),
        precision=jax.lax.Precision.DEFAULT,
        preferred_element_type=jnp.float32)


def _fused_body(x_ref, w1_ref, b1_ref, w2_ref, b2_ref, ow1_ref, ow2_ref,
                ob_ref, p_ref, o_ref):
    xb = x_ref[...]
    h = jnp.maximum(_dot(xb, w1_ref[...]) + b1_ref[...], 0.0)
    v = _dot(h, w2_ref[...]) + b2_ref[...]

    tv = p_ref[0:1, :]
    gb = p_ref[1:2, :]
    gc = p_ref[2:3, :]
    lw = p_ref[3:4, :]
    lb = p_ref[4:5, :]
    c0 = p_ref[5:6, :]
    c1 = p_ref[6:7, :]
    ra = p_ref[7:8, :]

    tri = jnp.maximum(v - jnp.abs(v - tv), 0.0)
    gau = jnp.exp((gb - (2.0 * _INV2SIG2) * v) * v + gc)
    lin = v * lw + lb
    dd = jnp.abs(v - c0) + jnp.abs(v - c1)
    da = jnp.abs(ra - dd)
    rat = (da - dd) / ((1.0 + dd) * (1.0 + da))
    acts = (tri + gau) + (lin + rat)

    o = _dot(xb, ow1_ref[...]) + _dot(acts, ow2_ref[...]) + ob_ref[...]
    o_ref[...] = jnp.maximum(o, 0.0)


@functools.partial(jax.jit, static_argnames=())
def kernel(x, edge_index, edge_slices, W1, b1, W2, b2, t_param, mu, line_W,
           line_b, c_param, r_param, out_W, out_b):
    del edge_index, edge_slices
    f32 = jnp.float32
    N, D = x.shape
    H = W1.shape[1]
    F = W2.shape[1]
    CF = t_param.shape[0]
    OUT = out_W.shape[1]
    K = 4 * CF
    A = F * K
    AP = 128

    W2R = jnp.pad(jnp.repeat(W2, K, axis=1), ((0, 0), (0, AP - A)))
    b2R = jnp.pad(jnp.repeat(b2, K), (0, AP - A)).reshape(1, AP)

    zc = jnp.zeros((CF,), f32)

    def col(a, b, c, d, fill=0.0):
        base = jnp.full((K,), fill, f32)
        base = base.at[0:CF].set(a).at[CF:2 * CF].set(b)
        base = base.at[2 * CF:3 * CF].set(c).at[3 * CF:4 * CF].set(d)
        return jnp.pad(jnp.tile(base, F), (0, AP - A), constant_values=fill)

    gb = 2.0 * _INV2SIG2 * (mu[:, 0] + mu[:, 1])
    gc = -_INV2SIG2 * (mu[:, 0] ** 2 + mu[:, 1] ** 2)
    neg_huge = jnp.full((CF,), -1e30, f32)
    P = jnp.stack([
        col(t_param, neg_huge, neg_huge, neg_huge, fill=-1e30),
        col(zc, gb, zc, zc),
        col(jnp.full((CF,), -1e38, f32), gc,
            jnp.full((CF,), -1e38, f32),
            jnp.full((CF,), -1e38, f32), fill=-1e38),
        col(zc, zc, line_W[0] + line_W[1], zc),
        col(zc, zc, line_b, zc),
        col(zc, zc, zc, c_param[:, 0]),
        col(zc, zc, zc, c_param[:, 1]),
        col(zc, zc, zc, jnp.full((CF,), jnp.abs(r_param[0]), f32)),
    ])

    oW1 = out_W[:D]
    oW2 = jnp.pad(out_W[D:], ((0, AP - A), (0, 0)))
    b1r = b1.reshape(1, H)
    obr = out_b.reshape(1, OUT)

    B = 5000
    grid = pl.cdiv(N, B)
    rep = lambda i: (0, 0)

    return pl.pallas_call(
        _fused_body,
        grid=(grid,),
        in_specs=[
            pl.BlockSpec((B, D), lambda i: (i, 0)),
            pl.BlockSpec((D, H), rep),
            pl.BlockSpec((1, H), rep),
            pl.BlockSpec((H, AP), rep),
            pl.BlockSpec((1, AP), rep),
            pl.BlockSpec((D, OUT), rep),
            pl.BlockSpec((AP, OUT), rep),
            pl.BlockSpec((1, OUT), rep),
            pl.BlockSpec((8, AP), rep),
        ],
        out_specs=pl.BlockSpec((B, OUT), lambda i: (i, 0)),
        out_shape=jax.ShapeDtypeStruct((N, OUT), f32),
        compiler_params=pltpu.CompilerParams(
            dimension_semantics=("parallel",)),
    )(x, W1, b1r, W2R, b2R, oW1, oW2, obr, P)
